# Initial kernel scaffold; baseline (speedup 1.0000x reference)
import jax
import jax.numpy as jnp
from jax.experimental import pallas as pl


def kernel(x, edge_index, edge_attr, batch, params):
    raise NotImplementedError("write your pallas kernel here")


# speedup vs baseline: 2.5466x; 2.5466x over previous
import functools
import math

import jax
import jax.numpy as jnp
from jax import lax
from jax.experimental import pallas as pl
from jax.experimental.pallas import tpu as pltpu
from jax.experimental.pallas import tpu_sc as plsc

F32 = jnp.float32
EPS = 1e-5

_NC = 2
_NS = 16
_NW = _NC * _NS
_CHUNK = 128
_BLKN = 512
_BLKE = 8192


def _sc_mesh():
    return plsc.VectorSubcoreMesh(
        core_axis_name="c", subcore_axis_name="s",
        num_cores=_NC, num_subcores=_NS)


def _mm(a, b):
    return lax.dot_general(a, b, (((1,), (0,)), ((), ())),
                           preferred_element_type=F32)


@functools.lru_cache(maxsize=None)
def _gather_fn(e_pad, d):
    cpt = e_pad // (_NW * _CHUNK)
    per_tile = cpt * _CHUNK

    def body(a_hbm, b_hbm, dst_hbm, src_hbm, ga_hbm, gb_hbm,
             idxd, idxs, ra, rb, sema, semb):
        c = lax.axis_index("c")
        s = lax.axis_index("s")
        wid = s * _NC + c
        base = wid * per_tile
        pltpu.sync_copy(dst_hbm.at[wid], idxd)
        pltpu.sync_copy(src_hbm.at[wid], idxs)

        def step(j, carry):
            off = base + j * _CHUNK
            cpa = pltpu.async_copy(a_hbm.at[idxd.at[j]], ra, sema)
            cpb = pltpu.async_copy(b_hbm.at[idxs.at[j]], rb, semb)
            cpa.wait()
            cpb.wait()
            pltpu.sync_copy(ra, ga_hbm.at[pl.ds(off, _CHUNK)])
            pltpu.sync_copy(rb, gb_hbm.at[pl.ds(off, _CHUNK)])
            return carry

        lax.fori_loop(0, cpt, step, 0)

    return pl.kernel(
        body,
        out_type=[jax.ShapeDtypeStruct((e_pad, d), F32),
                  jax.ShapeDtypeStruct((e_pad, d), F32)],
        mesh=_sc_mesh(),
        scratch_types=[pltpu.VMEM((cpt, _CHUNK), jnp.int32),
                       pltpu.VMEM((cpt, _CHUNK), jnp.int32),
                       pltpu.VMEM((_CHUNK, d), F32),
                       pltpu.VMEM((_CHUNK, d), F32),
                       pltpu.SemaphoreType.DMA,
                       pltpu.SemaphoreType.DMA],
        compiler_params=pltpu.CompilerParams(use_tc_tiling_on_sc=False),
    )


@functools.lru_cache(maxsize=None)
def _scatter_fn(n_pad, e_pad, d):
    cpt = e_pad // (_NW * _CHUNK)
    per_tile = cpt * _CHUNK
    rpt = n_pad // _NS

    def body(m_hbm, dst_hbm, zeros_hbm, out_hbm, idxd, mbuf, zbuf, shared):
        c = lax.axis_index("c")
        s = lax.axis_index("s")
        wid = s * _NC + c
        base = wid * per_tile
        r0 = s * rpt
        pltpu.sync_copy(dst_hbm.at[wid], idxd)
        pltpu.sync_copy(zeros_hbm.at[pl.ds(r0, rpt)], zbuf)
        pltpu.sync_copy(zbuf, shared.at[pl.ds(r0, rpt)])
        plsc.subcore_barrier()

        def step(j, carry):
            off = base + j * _CHUNK
            pltpu.sync_copy(m_hbm.at[pl.ds(off, _CHUNK)], mbuf)
            pltpu.sync_copy(mbuf, shared.at[idxd.at[j]], add=True)
            return carry

        lax.fori_loop(0, cpt, step, 0)
        plsc.subcore_barrier()
        pltpu.sync_copy(shared.at[pl.ds(r0, rpt)], zbuf)
        pltpu.sync_copy(zbuf, out_hbm.at[c, pl.ds(r0, rpt)])

    return pl.kernel(
        body,
        out_type=jax.ShapeDtypeStruct((_NC, n_pad, d), F32),
        mesh=_sc_mesh(),
        scratch_types=[pltpu.VMEM((cpt, _CHUNK), jnp.int32),
                       pltpu.VMEM((_CHUNK, d), F32),
                       pltpu.VMEM((rpt, d), F32),
                       pltpu.VMEM_SHARED((n_pad, d), F32)],
        compiler_params=pltpu.CompilerParams(use_tc_tiling_on_sc=False),
    )


def _full(shape):
    nd = len(shape)
    return pl.BlockSpec(shape, lambda i, _nd=nd: (0,) * _nd)


def _tc_project(x_p, winT, b_in, waT, wbT):
    n_pad, d_in = x_p.shape
    hid = winT.shape[1]
    dh = waT.shape[1]

    def body(x_ref, winT_ref, bin_ref, waT_ref, wbT_ref, h_ref, a_ref, b_ref):
        h = _mm(x_ref[...], winT_ref[...]) + bin_ref[...]
        h_ref[...] = h
        a_ref[...] = _mm(h, waT_ref[...])
        b_ref[...] = _mm(h, wbT_ref[...])

    grid = (n_pad // _BLKN,)
    return pl.pallas_call(
        body,
        grid=grid,
        in_specs=[pl.BlockSpec((_BLKN, d_in), lambda i: (i, 0)),
                  _full(winT.shape), _full(b_in.shape),
                  _full(waT.shape), _full(wbT.shape)],
        out_specs=[pl.BlockSpec((_BLKN, hid), lambda i: (i, 0)),
                   pl.BlockSpec((_BLKN, dh), lambda i: (i, 0)),
                   pl.BlockSpec((_BLKN, dh), lambda i: (i, 0))],
        out_shape=[jax.ShapeDtypeStruct((n_pad, hid), F32),
                   jax.ShapeDtypeStruct((n_pad, dh), F32),
                   jax.ShapeDtypeStruct((n_pad, dh), F32)],
    )(x_p, winT, b_in, waT, wbT)


def _tc_edge(ga, gb, c1, w2T, b2, wzaT, cza):
    e_pad, dh = ga.shape
    hid = w2T.shape[1]

    def body(ga_ref, gb_ref, c1_ref, w2T_ref, b2_ref, wzaT_ref, cza_ref,
             m_ref):
        u = jnp.maximum(ga_ref[...] + gb_ref[...] + c1_ref[...], 0.0)
        y = jnp.maximum(_mm(u, w2T_ref[...]) + b2_ref[...], 0.0)
        m_ref[...] = _mm(y, wzaT_ref[...]) + cza_ref[...]

    grid = (e_pad // _BLKE,)
    return pl.pallas_call(
        body,
        grid=grid,
        in_specs=[pl.BlockSpec((_BLKE, dh), lambda i: (i, 0)),
                  pl.BlockSpec((_BLKE, dh), lambda i: (i, 0)),
                  _full(c1.shape), _full(w2T.shape), _full(b2.shape),
                  _full(wzaT.shape), _full(cza.shape)],
        out_specs=pl.BlockSpec((_BLKE, dh), lambda i: (i, 0)),
        out_shape=jax.ShapeDtypeStruct((e_pad, dh), F32),
    )(ga, gb, c1, w2T, b2, wzaT, cza)


def _tc_update(h, p, wuhT, c1u, w2uT, b2u, s2u, be2u, waT, wbT):
    n_pad, hid = h.shape
    dh = wuhT.shape[1]

    def body(h_ref, p0_ref, p1_ref, wuhT_ref, c1u_ref, w2uT_ref, b2u_ref,
             s2u_ref, be2u_ref, waT_ref, wbT_ref, hn_ref, a_ref, b_ref):
        hv = h_ref[...]
        t = jnp.maximum(_mm(hv, wuhT_ref[...]) + p0_ref[...] + p1_ref[...]
                        + c1u_ref[...], 0.0)
        o = jnp.maximum(_mm(t, w2uT_ref[...]) + b2u_ref[...], 0.0)
        hn = hv + o * s2u_ref[...] + be2u_ref[...]
        hn_ref[...] = hn
        a_ref[...] = _mm(hn, waT_ref[...])
        b_ref[...] = _mm(hn, wbT_ref[...])

    grid = (n_pad // _BLKN,)
    return pl.pallas_call(
        body,
        grid=grid,
        in_specs=[pl.BlockSpec((_BLKN, hid), lambda i: (i, 0)),
                  pl.BlockSpec((_BLKN, dh), lambda i: (i, 0)),
                  pl.BlockSpec((_BLKN, dh), lambda i: (i, 0)),
                  _full(wuhT.shape), _full(c1u.shape), _full(w2uT.shape),
                  _full(b2u.shape), _full(s2u.shape), _full(be2u.shape),
                  _full(waT.shape), _full(wbT.shape)],
        out_specs=[pl.BlockSpec((_BLKN, hid), lambda i: (i, 0)),
                   pl.BlockSpec((_BLKN, dh), lambda i: (i, 0)),
                   pl.BlockSpec((_BLKN, dh), lambda i: (i, 0))],
        out_shape=[jax.ShapeDtypeStruct((n_pad, hid), F32),
                   jax.ShapeDtypeStruct((n_pad, dh), F32),
                   jax.ShapeDtypeStruct((n_pad, dh), F32)],
    )(h, p[0], p[1], wuhT, c1u, w2uT, b2u, s2u, be2u, waT, wbT)


def _tc_out(h, woutT, b_out):
    n_pad, hid = h.shape
    d_out = woutT.shape[1]

    def body(h_ref, woutT_ref, bout_ref, o_ref):
        o_ref[...] = _mm(h_ref[...], woutT_ref[...]) + bout_ref[...]

    grid = (n_pad // _BLKN,)
    return pl.pallas_call(
        body,
        grid=grid,
        in_specs=[pl.BlockSpec((_BLKN, hid), lambda i: (i, 0)),
                  _full(woutT.shape), _full(b_out.shape)],
        out_specs=pl.BlockSpec((_BLKN, d_out), lambda i: (i, 0)),
        out_shape=jax.ShapeDtypeStruct((n_pad, d_out), F32),
    )(h, woutT, b_out)


def _fold_layer(p_msg, p_upd):
    inv = 1.0 / jnp.sqrt(1.0 + EPS)
    s1 = p_msg["g1"] * inv
    s2 = p_msg["g2"] * inv
    s1u = p_upd["g1"] * inv
    hid = p_msg["W1"].shape[1] // 2
    w1d = p_msg["W1"][:, :hid]
    w1s = p_msg["W1"][:, hid:]
    w1uh = p_upd["W1"][:, :hid]
    w1ua = p_upd["W1"][:, hid:]
    return {
        "waT": w1d.T * s1[None, :],
        "wbT": w1s.T * s1[None, :],
        "c1": (p_msg["b1"] * s1 + p_msg["be1"])[None, :],
        "w2T": p_msg["W2"].T,
        "b2": p_msg["b2"][None, :],
        "wzaT": (s2[:, None] * w1ua.T) * s1u[None, :],
        "cza": ((p_msg["be2"] @ w1ua.T) * s1u)[None, :],
        "wuhT": w1uh.T * s1u[None, :],
        "c1u": (p_upd["b1"] * s1u + p_upd["be1"])[None, :],
        "w2uT": p_upd["W2"].T,
        "b2u": p_upd["b2"][None, :],
        "s2u": (p_upd["g2"] * inv)[None, :],
        "be2u": p_upd["be2"][None, :],
    }


def kernel(x, edge_index, edge_attr, batch, params):
    n, d_in = x.shape
    e = edge_index.shape[1]
    hid = params["W_in"].shape[0]
    num_layers = len(params["msg"])
    dh = params["msg"][0]["W1"].shape[0]

    n_pad = -(-n // _BLKN) * _BLKN
    step = _NW * _CHUNK
    quantum = step * _BLKE // math.gcd(step, _BLKE)
    e_pad = -(-e // quantum) * quantum
    cpt = e_pad // step

    x_p = jnp.pad(x, ((0, n_pad - n), (0, 0)))
    src = jnp.pad(edge_index[0], (0, e_pad - e)).reshape(_NW, cpt, _CHUNK)
    dst = jnp.pad(edge_index[1], (0, e_pad - e),
                  constant_values=n).reshape(_NW, cpt, _CHUNK)
    zeros = jnp.zeros((n_pad, dh), F32)
    folds = [_fold_layer(params["msg"][l], params["upd"][l])
             for l in range(num_layers)]

    gather = _gather_fn(e_pad, dh)
    scatter = _scatter_fn(n_pad, e_pad, dh)

    f0 = folds[0]
    h, a, b = _tc_project(x_p, params["W_in"].T, params["b_in"][None, :],
                          f0["waT"], f0["wbT"])
    for l in range(num_layers):
        f = folds[l]
        fn = folds[min(l + 1, num_layers - 1)]
        ga, gb = gather(a, b, dst, src)
        m = _tc_edge(ga, gb, f["c1"], f["w2T"], f["b2"], f["wzaT"], f["cza"])
        p = scatter(m, dst, zeros)
        h, a, b = _tc_update(h, p, f["wuhT"], f["c1u"], f["w2uT"], f["b2u"],
                             f["s2u"], f["be2u"], fn["waT"], fn["wbT"])

    out = _tc_out(h, params["W_out"].T, params["b_out"][None, :])
    return out[:n]
